# baseline (speedup 1.0000x reference)
import functools

import jax
import jax.numpy as jnp
from jax import lax
from jax.experimental import pallas as pl
from jax.experimental.pallas import tpu as pltpu
from jax.experimental.pallas import tpu_sc as plsc

N_NODES = 10000
N_EDGES = 320000
D = 128
DP = D // 2
N_HEADS = 8
ALPHA = 0.2

NP = 10240
NC = 2
NS = 16
B = 64
CH = 2
NBAT = 320
NCH = NBAT // CH
HPC = N_HEADS // NC
EPT = NBAT * B
E_PAD = EPT * NS
TROWS = NP // NS
ZR = 8


BN = 2000
NB = N_NODES // BN


def _proj_body(x_ref, w_ref, a_ref, h_ref, s1_ref, s2_ref):
    hd = pl.program_id(0)
    h = jnp.dot(x_ref[...], w_ref[0], preferred_element_type=jnp.float32)
    h_ref[0] = h
    av = a_ref[hd]
    s1_ref[0, 0, 0] = jnp.dot(h, av[:D], preferred_element_type=jnp.float32)
    s2_ref[0, 0, 0] = jnp.dot(h, av[D:], preferred_element_type=jnp.float32)


def _project(x, W, a):
    return pl.pallas_call(
        _proj_body,
        grid=(N_HEADS, NB),
        in_specs=[
            pl.BlockSpec((BN, D), lambda h, i: (i, 0)),
            pl.BlockSpec((1, D, D), lambda h, i: (h, 0, 0)),
            pl.BlockSpec((N_HEADS, 2 * D), lambda h, i: (0, 0)),
        ],
        out_specs=[
            pl.BlockSpec((1, BN, D), lambda h, i: (h, i, 0)),
            pl.BlockSpec((1, 1, 1, BN), lambda h, i: (h, i, 0, 0)),
            pl.BlockSpec((1, 1, 1, BN), lambda h, i: (h, i, 0, 0)),
        ],
        out_shape=[
            jax.ShapeDtypeStruct((N_HEADS, N_NODES, D), jnp.float32),
            jax.ShapeDtypeStruct((N_HEADS, NB, 1, BN), jnp.float32),
            jax.ShapeDtypeStruct((N_HEADS, NB, 1, BN), jnp.float32),
        ],
    )(x, W, a)


def _sc_body(h_hbm, s1_hbm, s2_hbm, src_hbm, dst_hbm, acc_out, rs_out,
             acc_sh, rs_sh, s1_v, s2_v,
             srcch0, srcch1, dstch0, dstch1,
             sidx0, sidx1, idx0, idx1, w0, w1,
             rows0, rows1, frows0, frows1,
             zbuf, z1_v,
             sem_g, sem_sr, sem_sw, sem_src, sem_dst):
    c = lax.axis_index("c")
    s = lax.axis_index("s")
    row_base = s * TROWS
    edge_base = s * EPT

    srcch = [srcch0, srcch1]
    dstch = [dstch0, dstch1]
    sidx = [sidx0, sidx1]
    idxb = [idx0, idx1]
    wb = [w0, w1]
    rows = [rows0, rows1]
    frows = [frows0, frows1]

    zeros16 = jnp.zeros((16,), jnp.float32)

    def _load_chunk(ci, q):
        base = edge_base + ci * (CH * B)
        pltpu.async_copy(src_hbm.at[pl.ds(base, CH * B)], srcch[q], sem_src.at[q])
        pltpu.async_copy(dst_hbm.at[pl.ds(base, CH * B)], dstch[q], sem_dst.at[q])

    def _head_body(hh, _):
        head = c * HPC + hh

        def _zb(i, _2):
            for k in range(D // 16):
                zbuf[i, pl.ds(k * 16, 16)] = zeros16
            return 0
        lax.fori_loop(0, ZR, _zb, 0)
        for g in range(TROWS // 16):
            z1_v[pl.ds(g * 16, 16)] = zeros16

        def _zacc(z, _2):
            pltpu.sync_copy(zbuf, acc_sh.at[pl.ds(row_base + z * ZR, ZR)])
            return 0
        lax.fori_loop(0, TROWS // ZR, _zacc, 0)
        pltpu.sync_copy(z1_v.at[pl.ds(0, TROWS)], rs_sh.at[pl.ds(row_base, TROWS)])

        pltpu.sync_copy(s1_hbm.at[pl.ds(head * NP, NP)], s1_v)
        pltpu.sync_copy(s2_hbm.at[pl.ds(head * NP, NP)], s2_v)
        plsc.subcore_barrier()

        _load_chunk(0, 0)

        def _issue(b, sub, q, p):
            for g in range(B // 16):
                sl = pl.ds(g * 16, 16)
                chsl = pl.ds(sub * B + g * 16, 16)
                si = srcch[q][chsl]
                di = dstch[q][chsl]
                t = plsc.load_gather(s1_v, [si]) + plsc.load_gather(s2_v, [di])
                wb[p][sl] = jnp.exp(jnp.minimum(-t, (-ALPHA) * t))
                sidx[p][sl] = si
                idxb[p][sl] = di + head * N_NODES
            pltpu.async_copy(h_hbm.at[idxb[p]], rows[p], sem_g.at[p])

        def _retire(p):
            pltpu.make_async_copy(h_hbm.at[idxb[p]], rows[p], sem_g.at[p]).wait()

            def _scale(g, _2):
                wg = wb[p][pl.ds(g * 16, 16)]
                for e in range(16):
                    ws = wg[e]
                    row = g * 16 + e
                    for k in range(DP // 16):
                        v = rows[p][row, pl.ds(k * 16, 16)]
                        vb = plsc.bitcast(v, jnp.bfloat16)
                        lo, hi = plsc.unpack(vb, format=plsc.PackFormat.INTERLEAVED)
                        frows[p][row, pl.ds(k * 32, 16)] = lo * ws
                        frows[p][row, pl.ds(k * 32 + 16, 16)] = hi * ws
                return 0
            lax.fori_loop(0, B // 16, _scale, 0)
            pltpu.async_copy(frows[p], acc_sh.at[sidx[p]], sem_sr.at[p], add=True)
            pltpu.async_copy(wb[p], rs_sh.at[sidx[p]], sem_sw.at[p], add=True)

        def _pair(cc, _2):
            for q in range(2):
                ci = 2 * cc + q
                @pl.when(ci + 1 < NCH)
                def _():
                    _load_chunk(ci + 1, 1 - q)
                pltpu.make_async_copy(
                    src_hbm.at[pl.ds(edge_base, CH * B)], srcch[q], sem_src.at[q]).wait()
                pltpu.make_async_copy(
                    dst_hbm.at[pl.ds(edge_base, CH * B)], dstch[q], sem_dst.at[q]).wait()
                for sub in range(CH):
                    b = ci * CH + sub
                    p = sub % 2
                    @pl.when(b >= 2)
                    def _():
                        pltpu.make_async_copy(
                            frows[p], acc_sh.at[sidx[p]], sem_sr.at[p]).wait()
                        pltpu.make_async_copy(
                            wb[p], rs_sh.at[sidx[p]], sem_sw.at[p]).wait()
                    _issue(b, sub, q, p)

                    @pl.when(b >= 1)
                    def _():
                        _retire(1 - p)
            return 0
        lax.fori_loop(0, NCH // 2, _pair, 0)

        _retire(1)
        for p in range(2):
            pltpu.make_async_copy(frows[p], acc_sh.at[sidx[p]], sem_sr.at[p]).wait()
            pltpu.make_async_copy(wb[p], rs_sh.at[sidx[p]], sem_sw.at[p]).wait()

        plsc.subcore_barrier()
        def _facc(z, _2):
            pltpu.sync_copy(acc_sh.at[pl.ds(row_base + z * ZR, ZR)], zbuf)
            pltpu.sync_copy(zbuf, acc_out.at[head, pl.ds(row_base + z * ZR, ZR)])
            return 0
        lax.fori_loop(0, TROWS // ZR, _facc, 0)
        pltpu.sync_copy(rs_sh.at[pl.ds(row_base, TROWS)], z1_v.at[pl.ds(0, TROWS)])
        pltpu.sync_copy(z1_v.at[pl.ds(0, TROWS)],
                        rs_out.at[pl.ds(head * NP + row_base, TROWS)])
        plsc.subcore_barrier()
        return 0

    lax.fori_loop(0, HPC, _head_body, 0)


_sc_edges = functools.partial(
    pl.kernel,
    out_type=(
        jax.ShapeDtypeStruct((N_HEADS, NP, D), jnp.float32),
        jax.ShapeDtypeStruct((N_HEADS * NP,), jnp.float32),
    ),
    mesh=plsc.VectorSubcoreMesh(core_axis_name="c", subcore_axis_name="s"),
    compiler_params=pltpu.CompilerParams(needs_layout_passes=False, use_tc_tiling_on_sc=False),
    scratch_types=[
        pltpu.VMEM_SHARED((NP, D), jnp.float32),
        pltpu.VMEM_SHARED((NP,), jnp.float32),
        pltpu.VMEM((NP,), jnp.float32),
        pltpu.VMEM((NP,), jnp.float32),
        pltpu.VMEM((CH * B,), jnp.int32),
        pltpu.VMEM((CH * B,), jnp.int32),
        pltpu.VMEM((CH * B,), jnp.int32),
        pltpu.VMEM((CH * B,), jnp.int32),
        pltpu.VMEM((B,), jnp.int32),
        pltpu.VMEM((B,), jnp.int32),
        pltpu.VMEM((B,), jnp.int32),
        pltpu.VMEM((B,), jnp.int32),
        pltpu.VMEM((B,), jnp.float32),
        pltpu.VMEM((B,), jnp.float32),
        pltpu.VMEM((B, DP), jnp.int32),
        pltpu.VMEM((B, DP), jnp.int32),
        pltpu.VMEM((B, D), jnp.float32),
        pltpu.VMEM((B, D), jnp.float32),
        pltpu.VMEM((ZR, D), jnp.float32),
        pltpu.VMEM((TROWS,), jnp.float32),
        pltpu.SemaphoreType.DMA((2,)),
        pltpu.SemaphoreType.DMA((2,)),
        pltpu.SemaphoreType.DMA((2,)),
        pltpu.SemaphoreType.DMA((2,)),
        pltpu.SemaphoreType.DMA((2,)),
    ],
)(_sc_body)


BR = 512


def _fin_body(acc_ref, rs_ref, o_ref):
    for h in range(N_HEADS):
        o_ref[:, h * D:(h + 1) * D] = acc_ref[h] / rs_ref[h][:, None]


def _finalize(acc, rs):
    return pl.pallas_call(
        _fin_body,
        grid=(pl.cdiv(N_NODES, BR),),
        in_specs=[
            pl.BlockSpec((N_HEADS, BR, D), lambda i: (0, i, 0)),
            pl.BlockSpec((N_HEADS, BR), lambda i: (0, i)),
        ],
        out_specs=pl.BlockSpec((BR, N_HEADS * D), lambda i: (i, 0)),
        out_shape=jax.ShapeDtypeStruct((N_NODES, N_HEADS * D), jnp.float32),
    )(acc, rs)


def kernel(x, edge_index, W, a):
    src = edge_index[0].astype(jnp.int32)
    dst = edge_index[1].astype(jnp.int32)
    pad = E_PAD - N_EDGES
    src = jnp.concatenate([src, jnp.full((pad,), N_NODES, jnp.int32)])
    dst = jnp.concatenate([dst, jnp.zeros((pad,), jnp.int32)])
    h, s1, s2 = _project(x, W, a)
    hb = h.astype(jnp.bfloat16).reshape(N_HEADS * N_NODES, DP // 16, 2, 16)
    hb = hb.swapaxes(2, 3)
    hpk = jax.lax.bitcast_convert_type(hb, jnp.int32)
    hpk = hpk.reshape(N_HEADS * N_NODES, DP)
    s1f = jnp.pad(s1.reshape(N_HEADS, N_NODES), ((0, 0), (0, NP - N_NODES)))
    s2f = jnp.pad(s2.reshape(N_HEADS, N_NODES), ((0, 0), (0, NP - N_NODES)))
    acc, rs = _sc_edges(hpk, s1f.reshape(-1), s2f.reshape(-1), src, dst)
    return _finalize(acc, rs.reshape(N_HEADS, NP))


# speedup vs baseline: 1.0863x; 1.0863x over previous
import functools

import jax
import jax.numpy as jnp
from jax import lax
from jax.experimental import pallas as pl
from jax.experimental.pallas import tpu as pltpu
from jax.experimental.pallas import tpu_sc as plsc

N_NODES = 10000
N_EDGES = 320000
D = 128
DP = D // 2
N_HEADS = 8
ALPHA = 0.2

NP = 10240
NC = 2
NS = 16
B = 64
CH = 2
NBAT = 320
NCH = NBAT // CH
HPC = N_HEADS // NC
EPT = NBAT * B
E_PAD = EPT * NS
TROWS = NP // NS
ZR = 8


BN = 2000
NB = N_NODES // BN


def _proj_body(x_ref, w_ref, a_ref, h_ref, s1_ref, s2_ref):
    hd = pl.program_id(0)
    h = jnp.dot(x_ref[...], w_ref[0], preferred_element_type=jnp.float32)
    h_ref[0] = h
    av = a_ref[hd]
    s1_ref[0, 0, 0] = jnp.dot(h, av[:D], preferred_element_type=jnp.float32)
    s2_ref[0, 0, 0] = jnp.dot(h, av[D:], preferred_element_type=jnp.float32)


def _project(x, W, a):
    return pl.pallas_call(
        _proj_body,
        grid=(N_HEADS, NB),
        in_specs=[
            pl.BlockSpec((BN, D), lambda h, i: (i, 0)),
            pl.BlockSpec((1, D, D), lambda h, i: (h, 0, 0)),
            pl.BlockSpec((N_HEADS, 2 * D), lambda h, i: (0, 0)),
        ],
        out_specs=[
            pl.BlockSpec((1, BN, D), lambda h, i: (h, i, 0)),
            pl.BlockSpec((1, 1, 1, BN), lambda h, i: (h, i, 0, 0)),
            pl.BlockSpec((1, 1, 1, BN), lambda h, i: (h, i, 0, 0)),
        ],
        out_shape=[
            jax.ShapeDtypeStruct((N_HEADS, N_NODES, D), jnp.float32),
            jax.ShapeDtypeStruct((N_HEADS, NB, 1, BN), jnp.float32),
            jax.ShapeDtypeStruct((N_HEADS, NB, 1, BN), jnp.float32),
        ],
    )(x, W, a)


def _sc_body(h_hbm, s1_hbm, s2_hbm, src_hbm, dst_hbm, acc_out, rs_out,
             acc_sh, rs_sh, s1_v, s2_v,
             srcch0, srcch1, dstch0, dstch1,
             sidx0, sidx1, idx0, idx1, w0, w1,
             rows0, rows1,
             zbuf, z1_v,
             sem_g, sem_sr, sem_sw, sem_src, sem_dst):
    c = lax.axis_index("c")
    s = lax.axis_index("s")
    row_base = s * TROWS
    edge_base = s * EPT

    srcch = [srcch0, srcch1]
    dstch = [dstch0, dstch1]
    sidx = [sidx0, sidx1]
    idxb = [idx0, idx1]
    wb = [w0, w1]
    rows = [rows0, rows1]

    zeros16 = jnp.zeros((16,), jnp.float32)

    def _load_chunk(ci, q):
        base = edge_base + ci * (CH * B)
        pltpu.async_copy(src_hbm.at[pl.ds(base, CH * B)], srcch[q], sem_src.at[q])
        pltpu.async_copy(dst_hbm.at[pl.ds(base, CH * B)], dstch[q], sem_dst.at[q])

    def _head_body(hh, _):
        head = c * HPC + hh

        def _zb(i, _2):
            for k in range(D // 16):
                zbuf[i, pl.ds(k * 16, 16)] = zeros16
            return 0
        lax.fori_loop(0, ZR, _zb, 0)
        for g in range(TROWS // 16):
            z1_v[pl.ds(g * 16, 16)] = zeros16

        def _zacc(z, _2):
            pltpu.sync_copy(zbuf, acc_sh.at[pl.ds(row_base + z * ZR, ZR)])
            return 0
        lax.fori_loop(0, TROWS // ZR, _zacc, 0)
        pltpu.sync_copy(z1_v.at[pl.ds(0, TROWS)], rs_sh.at[pl.ds(row_base, TROWS)])

        pltpu.sync_copy(s1_hbm.at[pl.ds(head * NP, NP)], s1_v)
        pltpu.sync_copy(s2_hbm.at[pl.ds(head * NP, NP)], s2_v)
        plsc.subcore_barrier()

        _load_chunk(0, 0)

        def _issue(b, sub, q, p):
            for g in range(B // 16):
                sl = pl.ds(g * 16, 16)
                chsl = pl.ds(sub * B + g * 16, 16)
                si = srcch[q][chsl]
                di = dstch[q][chsl]
                t = plsc.load_gather(s1_v, [si]) + plsc.load_gather(s2_v, [di])
                wb[p][sl] = jnp.exp(jnp.minimum(-t, (-ALPHA) * t))
                sidx[p][sl] = si
                idxb[p][sl] = di + head * N_NODES
            pltpu.async_copy(h_hbm.at[idxb[p]], rows[p], sem_g.at[p])

        def _retire(p):
            pltpu.make_async_copy(h_hbm.at[idxb[p]], rows[p], sem_g.at[p]).wait()

            def _scale(g, _2):
                wg = wb[p][pl.ds(g * 16, 16)]
                for e in range(16):
                    ws = wg[e]
                    row = g * 16 + e
                    for k in range(D // 16):
                        ksl = pl.ds(k * 16, 16)
                        rows[p][row, ksl] = rows[p][row, ksl] * ws
                return 0
            lax.fori_loop(0, B // 16, _scale, 0)
            pltpu.async_copy(rows[p], acc_sh.at[sidx[p]], sem_sr.at[p], add=True)
            pltpu.async_copy(wb[p], rs_sh.at[sidx[p]], sem_sw.at[p], add=True)

        def _pair(cc, _2):
            for q in range(2):
                ci = 2 * cc + q
                @pl.when(ci + 1 < NCH)
                def _():
                    _load_chunk(ci + 1, 1 - q)
                pltpu.make_async_copy(
                    src_hbm.at[pl.ds(edge_base, CH * B)], srcch[q], sem_src.at[q]).wait()
                pltpu.make_async_copy(
                    dst_hbm.at[pl.ds(edge_base, CH * B)], dstch[q], sem_dst.at[q]).wait()
                for sub in range(CH):
                    b = ci * CH + sub
                    p = sub % 2
                    @pl.when(b >= 2)
                    def _():
                        pltpu.make_async_copy(
                            rows[p], acc_sh.at[sidx[p]], sem_sr.at[p]).wait()
                        pltpu.make_async_copy(
                            wb[p], rs_sh.at[sidx[p]], sem_sw.at[p]).wait()
                    _issue(b, sub, q, p)

                    @pl.when(b >= 1)
                    def _():
                        _retire(1 - p)
            return 0
        lax.fori_loop(0, NCH // 2, _pair, 0)

        _retire(1)
        for p in range(2):
            pltpu.make_async_copy(rows[p], acc_sh.at[sidx[p]], sem_sr.at[p]).wait()
            pltpu.make_async_copy(wb[p], rs_sh.at[sidx[p]], sem_sw.at[p]).wait()

        plsc.subcore_barrier()
        def _facc(z, _2):
            pltpu.sync_copy(acc_sh.at[pl.ds(row_base + z * ZR, ZR)], zbuf)
            pltpu.sync_copy(zbuf, acc_out.at[head, pl.ds(row_base + z * ZR, ZR)])
            return 0
        lax.fori_loop(0, TROWS // ZR, _facc, 0)
        pltpu.sync_copy(rs_sh.at[pl.ds(row_base, TROWS)], z1_v.at[pl.ds(0, TROWS)])
        pltpu.sync_copy(z1_v.at[pl.ds(0, TROWS)],
                        rs_out.at[pl.ds(head * NP + row_base, TROWS)])
        plsc.subcore_barrier()
        return 0

    lax.fori_loop(0, HPC, _head_body, 0)


_sc_edges = functools.partial(
    pl.kernel,
    out_type=(
        jax.ShapeDtypeStruct((N_HEADS, NP, D), jnp.float32),
        jax.ShapeDtypeStruct((N_HEADS * NP,), jnp.float32),
    ),
    mesh=plsc.VectorSubcoreMesh(core_axis_name="c", subcore_axis_name="s"),
    compiler_params=pltpu.CompilerParams(needs_layout_passes=False),
    scratch_types=[
        pltpu.VMEM_SHARED((NP, D), jnp.float32),
        pltpu.VMEM_SHARED((NP,), jnp.float32),
        pltpu.VMEM((NP,), jnp.float32),
        pltpu.VMEM((NP,), jnp.float32),
        pltpu.VMEM((CH * B,), jnp.int32),
        pltpu.VMEM((CH * B,), jnp.int32),
        pltpu.VMEM((CH * B,), jnp.int32),
        pltpu.VMEM((CH * B,), jnp.int32),
        pltpu.VMEM((B,), jnp.int32),
        pltpu.VMEM((B,), jnp.int32),
        pltpu.VMEM((B,), jnp.int32),
        pltpu.VMEM((B,), jnp.int32),
        pltpu.VMEM((B,), jnp.float32),
        pltpu.VMEM((B,), jnp.float32),
        pltpu.VMEM((B, D), jnp.float32),
        pltpu.VMEM((B, D), jnp.float32),
        pltpu.VMEM((ZR, D), jnp.float32),
        pltpu.VMEM((TROWS,), jnp.float32),
        pltpu.SemaphoreType.DMA((2,)),
        pltpu.SemaphoreType.DMA((2,)),
        pltpu.SemaphoreType.DMA((2,)),
        pltpu.SemaphoreType.DMA((2,)),
        pltpu.SemaphoreType.DMA((2,)),
    ],
)(_sc_body)


BR = 512


def _fin_body(acc_ref, rs_ref, o_ref):
    for h in range(N_HEADS):
        o_ref[:, h * D:(h + 1) * D] = acc_ref[h] / rs_ref[h][:, None]


def _finalize(acc, rs):
    return pl.pallas_call(
        _fin_body,
        grid=(pl.cdiv(N_NODES, BR),),
        in_specs=[
            pl.BlockSpec((N_HEADS, BR, D), lambda i: (0, i, 0)),
            pl.BlockSpec((N_HEADS, BR), lambda i: (0, i)),
        ],
        out_specs=pl.BlockSpec((BR, N_HEADS * D), lambda i: (i, 0)),
        out_shape=jax.ShapeDtypeStruct((N_NODES, N_HEADS * D), jnp.float32),
    )(acc, rs)


def kernel(x, edge_index, W, a):
    src = edge_index[0].astype(jnp.int32)
    dst = edge_index[1].astype(jnp.int32)
    pad = E_PAD - N_EDGES
    src = jnp.concatenate([src, jnp.full((pad,), N_NODES, jnp.int32)])
    dst = jnp.concatenate([dst, jnp.zeros((pad,), jnp.int32)])
    h, s1, s2 = _project(x, W, a)
    hpk = h.reshape(N_HEADS * N_NODES, D)
    s1f = jnp.pad(s1.reshape(N_HEADS, N_NODES), ((0, 0), (0, NP - N_NODES)))
    s2f = jnp.pad(s2.reshape(N_HEADS, N_NODES), ((0, 0), (0, NP - N_NODES)))
    acc, rs = _sc_edges(hpk, s1f.reshape(-1), s2f.reshape(-1), src, dst)
    return _finalize(acc, rs.reshape(N_HEADS, NP))


# speedup vs baseline: 1.5264x; 1.4051x over previous
import functools

import jax
import jax.numpy as jnp
from jax import lax
from jax.experimental import pallas as pl
from jax.experimental.pallas import tpu as pltpu
from jax.experimental.pallas import tpu_sc as plsc

N_NODES = 10000
N_EDGES = 320000
D = 128
DP = D // 2
N_HEADS = 8
ALPHA = 0.2

NP = 10240
NC = 2
NS = 16
B = 64
CH = 2
NBAT = 316
NCH = NBAT // CH
HPC = N_HEADS // NC
EPT = NBAT * B
E_PAD = EPT * NS
TROWS = NP // NS
ZR = 40


BN = 2000
NB = N_NODES // BN


def _proj_body(x_ref, w_ref, a_ref, h_ref, s1_ref, s2_ref):
    hd = pl.program_id(0)
    h = jnp.dot(x_ref[...], w_ref[0], preferred_element_type=jnp.float32)
    h_ref[0] = h
    av = a_ref[hd]
    s1_ref[0, 0, 0] = jnp.dot(h, av[:D], preferred_element_type=jnp.float32)
    s2_ref[0, 0, 0] = jnp.dot(h, av[D:], preferred_element_type=jnp.float32)


def _project(x, W, a):
    return pl.pallas_call(
        _proj_body,
        grid=(N_HEADS, NB),
        in_specs=[
            pl.BlockSpec((BN, D), lambda h, i: (i, 0)),
            pl.BlockSpec((1, D, D), lambda h, i: (h, 0, 0)),
            pl.BlockSpec((N_HEADS, 2 * D), lambda h, i: (0, 0)),
        ],
        out_specs=[
            pl.BlockSpec((1, BN, D), lambda h, i: (h, i, 0)),
            pl.BlockSpec((1, 1, 1, BN), lambda h, i: (h, i, 0, 0)),
            pl.BlockSpec((1, 1, 1, BN), lambda h, i: (h, i, 0, 0)),
        ],
        out_shape=[
            jax.ShapeDtypeStruct((N_HEADS, N_NODES, D), jnp.float32),
            jax.ShapeDtypeStruct((N_HEADS, NB, 1, BN), jnp.float32),
            jax.ShapeDtypeStruct((N_HEADS, NB, 1, BN), jnp.float32),
        ],
    )(x, W, a)


def _sc_body(h_hbm, s1_hbm, s2_hbm, src_hbm, dst_hbm, acc_out, rs_out,
             acc_sh, rs_sh, s1_v, s2_v,
             srcch0, srcch1, dstch0, dstch1,
             sidx0, sidx1, idx0, idx1, w0, w1,
             rows0, rows1,
             zbuf, z1_v,
             sem_g, sem_sr, sem_sw, sem_src, sem_dst):
    c = lax.axis_index("c")
    s = lax.axis_index("s")
    row_base = s * TROWS
    edge_base = s * EPT

    srcch = [srcch0, srcch1]
    dstch = [dstch0, dstch1]
    sidx = [sidx0, sidx1]
    idxb = [idx0, idx1]
    wb = [w0, w1]
    rows = [rows0, rows1]

    zeros16 = jnp.zeros((16,), jnp.float32)

    def _load_chunk(ci, q):
        base = edge_base + ci * (CH * B)
        pltpu.async_copy(src_hbm.at[pl.ds(base, CH * B)], srcch[q], sem_src.at[q])
        pltpu.async_copy(dst_hbm.at[pl.ds(base, CH * B)], dstch[q], sem_dst.at[q])

    def _head_body(hh, _):
        head = c * HPC + hh

        def _zb(i, _2):
            for k in range(D // 16):
                zbuf[i, pl.ds(k * 16, 16)] = zeros16
            return 0
        lax.fori_loop(0, ZR, _zb, 0)
        for g in range(TROWS // 16):
            z1_v[pl.ds(g * 16, 16)] = zeros16

        def _zacc(z, _2):
            pltpu.sync_copy(zbuf, acc_sh.at[pl.ds(row_base + z * ZR, ZR)])
            return 0
        lax.fori_loop(0, TROWS // ZR, _zacc, 0)
        pltpu.sync_copy(z1_v.at[pl.ds(0, TROWS)], rs_sh.at[pl.ds(row_base, TROWS)])

        pltpu.sync_copy(s1_hbm.at[pl.ds(head * NP, NP)], s1_v)
        pltpu.sync_copy(s2_hbm.at[pl.ds(head * NP, NP)], s2_v)
        plsc.subcore_barrier()

        _load_chunk(0, 0)

        def _issue(b, sub, q, p):
            for g in range(B // 16):
                sl = pl.ds(g * 16, 16)
                chsl = pl.ds(sub * B + g * 16, 16)
                si = srcch[q][chsl]
                di = dstch[q][chsl]
                t = plsc.load_gather(s1_v, [si]) + plsc.load_gather(s2_v, [di])
                wb[p][sl] = jnp.exp(jnp.minimum(-t, (-ALPHA) * t))
                sidx[p][sl] = si
                idxb[p][sl] = di + head * N_NODES
            pltpu.async_copy(h_hbm.at[idxb[p]], rows[p], sem_g.at[p])

        def _retire(p):
            pltpu.make_async_copy(h_hbm.at[idxb[p]], rows[p], sem_g.at[p]).wait()

            def _scale(g, _2):
                wg = wb[p][pl.ds(g * 16, 16)]
                for e in range(16):
                    ws = wg[e]
                    row = g * 16 + e
                    for k in range(D // 16):
                        ksl = pl.ds(k * 16, 16)
                        rows[p][row, ksl] = rows[p][row, ksl] * ws
                return 0
            lax.fori_loop(0, B // 16, _scale, 0)
            pltpu.async_copy(rows[p], acc_sh.at[sidx[p]], sem_sr.at[p], add=True)
            pltpu.async_copy(wb[p], rs_sh.at[sidx[p]], sem_sw.at[p], add=True)

        def _pair(cc, _2):
            for q in range(2):
                ci = 2 * cc + q
                @pl.when(ci + 1 < NCH)
                def _():
                    _load_chunk(ci + 1, 1 - q)
                pltpu.make_async_copy(
                    src_hbm.at[pl.ds(edge_base, CH * B)], srcch[q], sem_src.at[q]).wait()
                pltpu.make_async_copy(
                    dst_hbm.at[pl.ds(edge_base, CH * B)], dstch[q], sem_dst.at[q]).wait()
                for sub in range(CH):
                    b = ci * CH + sub
                    p = sub % 2
                    @pl.when(b >= 2)
                    def _():
                        pltpu.make_async_copy(
                            rows[p], acc_sh.at[sidx[p]], sem_sr.at[p]).wait()
                        pltpu.make_async_copy(
                            wb[p], rs_sh.at[sidx[p]], sem_sw.at[p]).wait()
                    _issue(b, sub, q, p)

                    @pl.when(b >= 1)
                    def _():
                        _retire(1 - p)
            return 0
        lax.fori_loop(0, NCH // 2, _pair, 0)

        _retire(1)
        for p in range(2):
            pltpu.make_async_copy(rows[p], acc_sh.at[sidx[p]], sem_sr.at[p]).wait()
            pltpu.make_async_copy(wb[p], rs_sh.at[sidx[p]], sem_sw.at[p]).wait()

        plsc.subcore_barrier()
        def _facc(z, _2):
            pltpu.sync_copy(acc_sh.at[pl.ds(row_base + z * ZR, ZR)], zbuf)
            pltpu.sync_copy(zbuf, acc_out.at[head, pl.ds(row_base + z * ZR, ZR)])
            return 0
        lax.fori_loop(0, TROWS // ZR, _facc, 0)
        pltpu.sync_copy(rs_sh.at[pl.ds(row_base, TROWS)], z1_v.at[pl.ds(0, TROWS)])
        pltpu.sync_copy(z1_v.at[pl.ds(0, TROWS)],
                        rs_out.at[pl.ds(head * NP + row_base, TROWS)])
        plsc.subcore_barrier()
        return 0

    lax.fori_loop(0, HPC, _head_body, 0)


_sc_edges = functools.partial(
    pl.kernel,
    out_type=(
        jax.ShapeDtypeStruct((N_HEADS, NP, D), jnp.float32),
        jax.ShapeDtypeStruct((N_HEADS * NP,), jnp.float32),
    ),
    mesh=plsc.VectorSubcoreMesh(core_axis_name="c", subcore_axis_name="s"),
    compiler_params=pltpu.CompilerParams(needs_layout_passes=False),
    scratch_types=[
        pltpu.VMEM_SHARED((NP, D), jnp.float32),
        pltpu.VMEM_SHARED((NP,), jnp.float32),
        pltpu.VMEM((NP,), jnp.float32),
        pltpu.VMEM((NP,), jnp.float32),
        pltpu.VMEM((CH * B,), jnp.int32),
        pltpu.VMEM((CH * B,), jnp.int32),
        pltpu.VMEM((CH * B,), jnp.int32),
        pltpu.VMEM((CH * B,), jnp.int32),
        pltpu.VMEM((B,), jnp.int32),
        pltpu.VMEM((B,), jnp.int32),
        pltpu.VMEM((B,), jnp.int32),
        pltpu.VMEM((B,), jnp.int32),
        pltpu.VMEM((B,), jnp.float32),
        pltpu.VMEM((B,), jnp.float32),
        pltpu.VMEM((B, D), jnp.float32),
        pltpu.VMEM((B, D), jnp.float32),
        pltpu.VMEM((ZR, D), jnp.float32),
        pltpu.VMEM((TROWS,), jnp.float32),
        pltpu.SemaphoreType.DMA((2,)),
        pltpu.SemaphoreType.DMA((2,)),
        pltpu.SemaphoreType.DMA((2,)),
        pltpu.SemaphoreType.DMA((2,)),
        pltpu.SemaphoreType.DMA((2,)),
    ],
)(_sc_body)


BR = 512


def _fin_body(acc_ref, rs_ref, o_ref):
    for h in range(N_HEADS):
        o_ref[:, h * D:(h + 1) * D] = acc_ref[h] / rs_ref[h][:, None]


def _finalize(acc, rs):
    return pl.pallas_call(
        _fin_body,
        grid=(pl.cdiv(N_NODES, BR),),
        in_specs=[
            pl.BlockSpec((N_HEADS, BR, D), lambda i: (0, i, 0)),
            pl.BlockSpec((N_HEADS, BR), lambda i: (0, i)),
        ],
        out_specs=pl.BlockSpec((BR, N_HEADS * D), lambda i: (i, 0)),
        out_shape=jax.ShapeDtypeStruct((N_NODES, N_HEADS * D), jnp.float32),
    )(acc, rs)


def kernel(x, edge_index, W, a):
    src = edge_index[0].astype(jnp.int32)
    dst = edge_index[1].astype(jnp.int32)
    pad = E_PAD - N_EDGES
    src = jnp.concatenate([src, jnp.full((pad,), N_NODES, jnp.int32)])
    dst = jnp.concatenate([dst, jnp.zeros((pad,), jnp.int32)])
    h, s1, s2 = _project(x, W, a)
    hpk = h.reshape(N_HEADS * N_NODES, D)
    s1f = jnp.pad(s1.reshape(N_HEADS, N_NODES), ((0, 0), (0, NP - N_NODES)))
    s2f = jnp.pad(s2.reshape(N_HEADS, N_NODES), ((0, 0), (0, NP - N_NODES)))
    acc, rs = _sc_edges(hpk, s1f.reshape(-1), s2f.reshape(-1), src, dst)
    return _finalize(acc, rs.reshape(N_HEADS, NP))
